# baseline (speedup 1.0000x reference)
import functools

import jax
import jax.numpy as jnp
from jax.experimental import pallas as pl
from jax.experimental.pallas import tpu as pltpu

D_MODEL = 2048
TOT_EXPERT = 64
TOP_K = 8
ALPHA = 1.0

BLOCK_R = 512


def _router_block(w_ref, x_ref, avg_ref, idx_ref, score_ref):
    w = w_ref[...]
    x = x_ref[...]
    logits = jax.lax.dot_general(
        w, x,
        dimension_numbers=(((1,), (1,)), ((), ())),
        preferred_element_type=jnp.float32,
    )
    vals = logits - ALPHA * avg_ref[...]

    row = jax.lax.broadcasted_iota(jnp.int32, vals.shape, 0)
    top_vals = []
    top_idx = []
    for _ in range(TOP_K):
        m = jnp.max(vals, axis=0, keepdims=True)
        eq = vals == m
        i = jnp.min(jnp.where(eq, row, TOT_EXPERT), axis=0, keepdims=True)
        top_vals.append(m)
        top_idx.append(i)
        vals = jnp.where(row == i, -jnp.inf, vals)

    tv = jnp.concatenate(top_vals, axis=0)
    ti = jnp.concatenate(top_idx, axis=0)
    e = jnp.exp(tv - tv[0:1, :])
    s = e / jnp.sum(e, axis=0, keepdims=True)
    idx_ref[...] = ti.T
    score_ref[...] = s.T


@functools.partial(jax.jit, static_argnames=())
def kernel(inp, W, avg_logits):
    n = inp.shape[0]
    grid = (n // BLOCK_R,)
    avg2 = avg_logits.reshape(TOT_EXPERT, 1)
    out_idx, out_score = pl.pallas_call(
        _router_block,
        grid=grid,
        in_specs=[
            pl.BlockSpec((TOT_EXPERT, D_MODEL), lambda i: (0, 0)),
            pl.BlockSpec((BLOCK_R, D_MODEL), lambda i: (i, 0)),
            pl.BlockSpec((TOT_EXPERT, 1), lambda i: (0, 0)),
        ],
        out_specs=[
            pl.BlockSpec((BLOCK_R, TOP_K), lambda i: (i, 0)),
            pl.BlockSpec((BLOCK_R, TOP_K), lambda i: (i, 0)),
        ],
        out_shape=[
            jax.ShapeDtypeStruct((n, TOP_K), jnp.int32),
            jax.ShapeDtypeStruct((n, TOP_K), jnp.float32),
        ],
        compiler_params=pltpu.CompilerParams(
            dimension_semantics=("parallel",),
        ),
    )(W, inp, avg2)
    return (out_idx, out_score)


# speedup vs baseline: 1.1859x; 1.1859x over previous
import functools

import jax
import jax.numpy as jnp
from jax.experimental import pallas as pl
from jax.experimental.pallas import tpu as pltpu

D_MODEL = 2048
TOT_EXPERT = 64
TOP_K = 8
ALPHA = 1.0

BLOCK_R = 1024


def _router_block(w_ref, x_ref, avg_ref, idx_ref, score_ref):
    w = w_ref[...]
    x = x_ref[...]
    logits = jax.lax.dot_general(
        w, x,
        dimension_numbers=(((1,), (1,)), ((), ())),
        preferred_element_type=jnp.float32,
    )
    vals = logits - ALPHA * avg_ref[...]

    row = jax.lax.broadcasted_iota(jnp.int32, vals.shape, 0)
    top_vals = []
    top_idx = []
    for _ in range(TOP_K):
        m = jnp.max(vals, axis=0, keepdims=True)
        eq = vals == m
        i = jnp.min(jnp.where(eq, row, TOT_EXPERT), axis=0, keepdims=True)
        top_vals.append(m)
        top_idx.append(i)
        vals = jnp.where(row == i, -jnp.inf, vals)

    tv = jnp.concatenate(top_vals, axis=0)
    ti = jnp.concatenate(top_idx, axis=0)
    e = jnp.exp(tv - tv[0:1, :])
    s = e / jnp.sum(e, axis=0, keepdims=True)
    idx_ref[...] = ti.T
    score_ref[...] = s.T


@functools.partial(jax.jit, static_argnames=())
def kernel(inp, W, avg_logits):
    n = inp.shape[0]
    grid = (n // BLOCK_R,)
    avg2 = avg_logits.reshape(TOT_EXPERT, 1)
    out_idx, out_score = pl.pallas_call(
        _router_block,
        grid=grid,
        in_specs=[
            pl.BlockSpec((TOT_EXPERT, D_MODEL), lambda i: (0, 0)),
            pl.BlockSpec((BLOCK_R, D_MODEL), lambda i: (i, 0)),
            pl.BlockSpec((TOT_EXPERT, 1), lambda i: (0, 0)),
        ],
        out_specs=[
            pl.BlockSpec((BLOCK_R, TOP_K), lambda i: (i, 0)),
            pl.BlockSpec((BLOCK_R, TOP_K), lambda i: (i, 0)),
        ],
        out_shape=[
            jax.ShapeDtypeStruct((n, TOP_K), jnp.int32),
            jax.ShapeDtypeStruct((n, TOP_K), jnp.float32),
        ],
        compiler_params=pltpu.CompilerParams(
            dimension_semantics=("parallel",),
        ),
    )(W, inp, avg2)
    return (out_idx, out_score)


# speedup vs baseline: 1.2675x; 1.0688x over previous
import functools

import jax
import jax.numpy as jnp
from jax.experimental import pallas as pl
from jax.experimental.pallas import tpu as pltpu

D_MODEL = 2048
TOT_EXPERT = 64
TOP_K = 8
ALPHA = 1.0

BLOCK_R = 2048


def _router_block(w_ref, x_ref, avg_ref, idx_ref, score_ref):
    w = w_ref[...]
    x = x_ref[...]
    logits = jax.lax.dot_general(
        w, x,
        dimension_numbers=(((1,), (1,)), ((), ())),
        preferred_element_type=jnp.float32,
    )
    vals = logits - ALPHA * avg_ref[...]

    row = jax.lax.broadcasted_iota(jnp.int32, vals.shape, 0)
    top_vals = []
    top_idx = []
    for _ in range(TOP_K):
        m = jnp.max(vals, axis=0, keepdims=True)
        eq = vals == m
        i = jnp.min(jnp.where(eq, row, TOT_EXPERT), axis=0, keepdims=True)
        top_vals.append(m)
        top_idx.append(i)
        vals = jnp.where(row == i, -jnp.inf, vals)

    tv = jnp.concatenate(top_vals, axis=0)
    ti = jnp.concatenate(top_idx, axis=0)
    e = jnp.exp(tv - tv[0:1, :])
    s = e / jnp.sum(e, axis=0, keepdims=True)
    idx_ref[...] = ti.T
    score_ref[...] = s.T


@functools.partial(jax.jit, static_argnames=())
def kernel(inp, W, avg_logits):
    n = inp.shape[0]
    grid = (n // BLOCK_R,)
    avg2 = avg_logits.reshape(TOT_EXPERT, 1)
    out_idx, out_score = pl.pallas_call(
        _router_block,
        grid=grid,
        in_specs=[
            pl.BlockSpec((TOT_EXPERT, D_MODEL), lambda i: (0, 0)),
            pl.BlockSpec((BLOCK_R, D_MODEL), lambda i: (i, 0)),
            pl.BlockSpec((TOT_EXPERT, 1), lambda i: (0, 0)),
        ],
        out_specs=[
            pl.BlockSpec((BLOCK_R, TOP_K), lambda i: (i, 0)),
            pl.BlockSpec((BLOCK_R, TOP_K), lambda i: (i, 0)),
        ],
        out_shape=[
            jax.ShapeDtypeStruct((n, TOP_K), jnp.int32),
            jax.ShapeDtypeStruct((n, TOP_K), jnp.float32),
        ],
        compiler_params=pltpu.CompilerParams(
            dimension_semantics=("parallel",),
        ),
    )(W, inp, avg2)
    return (out_idx, out_score)
